# baseline (speedup 1.0000x reference)
import jax
import jax.numpy as jnp
from jax import lax
from jax.experimental import pallas as pl
from jax.experimental.pallas import tpu as pltpu

_CROSS_AVG_ERR = 0.25
_CROSS_WEIGHT = 1.0
_MIN_SUPPORT = 20.0
_MAX_BEAR = 20.0

_B = 8
_NP = 2048
_NF = 2048
_KP = 16
_D = 128
_NCHUNK = 4
_QC = _NP // _NCHUNK


def _tc_chamfer_kernel(pt_ref, yt_ref, segp_ref, prm_ref, sd_ref):
    pt = pt_ref[0]
    px, py, pz = pt[0:1, :], pt[1:2, :], pt[2:3, :]
    m = [prm_ref[0, 0, j] for j in range(12)]
    tx = px * m[0] + py * m[3] + pz * m[6] + m[9]
    ty = px * m[1] + py * m[4] + pz * m[7] + m[10]
    tz = px * m[2] + py * m[5] + pz * m[8] + m[11]
    xn2 = tx * tx + ty * ty + tz * tz
    x4 = jnp.concatenate([-2.0 * tx, -2.0 * ty, -2.0 * tz,
                          jnp.ones((1, _NP), jnp.float32)], axis=0)
    yr = yt_ref[0]
    yn2 = (yr[0:1, :] * yr[0:1, :] + yr[1:2, :] * yr[1:2, :]
           + yr[2:3, :] * yr[2:3, :])
    y4t = jnp.concatenate([yr, yn2], axis=0)
    d = jax.lax.dot_general(
        y4t, x4, dimension_numbers=(((0,), (0,)), ((), ())),
        preferred_element_type=jnp.float32)
    cham = jnp.min(d, axis=0, keepdims=True) + xn2
    chamc = jnp.transpose(cham, (1, 0))
    segp = segp_ref[0]
    sd = jnp.sum(segp * chamc, axis=0, keepdims=True)
    sd_ref[...] = sd.reshape(1, 1, _KP)


def _tc_final_kernel(sd_ref, segp_ref, segf_ref, rtf_ref, rtp_ref,
                     out_ref):
    seg2def = sd_ref[:, 0, :]
    kp_p = jnp.concatenate(
        [jnp.sum(segp_ref[b], axis=0, keepdims=True) for b in range(_B)],
        axis=0)
    kp_f = jnp.concatenate(
        [jnp.sum(segf_ref[b], axis=0, keepdims=True) for b in range(_B)],
        axis=0)
    seg_def = seg2def / kp_p
    rel = jax.nn.sigmoid(seg_def / _CROSS_AVG_ERR)
    rfn = rtf_ref[...] / kp_f[..., None]
    rpn = rtp_ref[...] / kp_p[..., None]
    diff = rfn - rpn
    r_dis = jnp.sum(diff * diff, axis=-1)
    loss_rd = (r_dis - rel) ** 2
    mask = ((kp_p >= _MIN_SUPPORT) & (kp_f >= _MIN_SUPPORT)
            & (loss_rd <= _MAX_BEAR))
    maskf = mask.astype(jnp.float32)
    nofL = jnp.sum(maskf)
    total = jnp.sum(loss_rd * maskf) / (nofL + 1.0) * _CROSS_WEIGHT
    out_ref[...] = jnp.full((1, 1), jnp.where(nofL == 0.0, 0.0, total))


@jax.jit
def kernel(r_tokens_full, r_tokens_partial, pc_seg_full, pc_seg_partial,
           recon_pc_full, recon_pc_partial, deformed,
           rot_full, t_full, rot_partial, t_partial,
           tgt_rand_rot, tgt_rand_t, src_rand_rot, src_rand_t):
    del recon_pc_full
    r2t = jnp.transpose(tgt_rand_rot, (0, 2, 1))
    r4t = jnp.transpose(rot_full, (0, 2, 1))
    mmat = rot_partial @ r2t @ src_rand_rot @ r4t
    cvec = ((t_partial - tgt_rand_t) @ r2t @ src_rand_rot
            + src_rand_t - t_full) @ r4t
    params = jnp.concatenate(
        [mmat.reshape(_B, 9), cvec.reshape(_B, 3)], axis=1)
    params = params.reshape(_B, 1, 12)
    pt = jnp.transpose(recon_pc_partial, (0, 2, 1))

    sd = pl.pallas_call(
        _tc_chamfer_kernel,
        grid=(_B,),
        in_specs=[
            pl.BlockSpec((1, 3, _NP), lambda b: (b, 0, 0)),
            pl.BlockSpec((1, 3, _NF), lambda b: (b, 0, 0)),
            pl.BlockSpec((1, _NP, _KP), lambda b: (b, 0, 0)),
            pl.BlockSpec((1, 1, 12), lambda b: (b, 0, 0),
                         memory_space=pltpu.SMEM),
        ],
        out_specs=pl.BlockSpec((1, 1, _KP), lambda b: (b, 0, 0)),
        out_shape=jax.ShapeDtypeStruct((_B, 1, _KP), jnp.float32),
        compiler_params=pltpu.CompilerParams(
            dimension_semantics=("parallel",)),
    )(pt, deformed, pc_seg_partial, params)

    out = pl.pallas_call(
        _tc_final_kernel,
        out_shape=jax.ShapeDtypeStruct((1, 1), jnp.float32),
    )(sd, pc_seg_partial, pc_seg_full, r_tokens_full, r_tokens_partial)
    return out[0, 0]
